# Initial kernel scaffold; baseline (speedup 1.0000x reference)
import jax
import jax.numpy as jnp
from jax.experimental import pallas as pl


def kernel(x, codebook, gumbel_noise):
    raise NotImplementedError("write your pallas kernel here")


# speedup vs baseline: 5.7809x; 5.7809x over previous
import jax
import jax.numpy as jnp
from jax.experimental import pallas as pl
from jax.experimental.pallas import tpu as pltpu

B, N, D, K = 8, 576, 64, 512
T = 512
G = (B * N) // T


def _vq_tile(x_ref, cb_ref, g_ref, q_ref, idx_ref, enc_ref):
    x = x_ref[...]
    cb = cb_ref[...]
    g = g_ref[...]

    xn2 = jnp.sum(x * x, axis=1, keepdims=True)
    cn2 = jnp.sum(cb * cb, axis=1)[None, :]
    xc = jax.lax.dot_general(
        x, cb, (((1,), (1,)), ((), ())),
        precision=jax.lax.Precision.HIGHEST,
        preferred_element_type=jnp.float32)
    d2 = xn2 - 2.0 * xc + cn2
    d = jnp.sqrt(jnp.maximum(d2, 0.0))

    dmin = jnp.min(d, axis=1, keepdims=True)
    iota = jax.lax.broadcasted_iota(jnp.int32, (T, K), 1)
    idx = jnp.min(jnp.where(d == dmin, iota, K), axis=1)
    idx_ref[0, 0, :] = idx

    logits = g - d
    m = jnp.max(logits, axis=1, keepdims=True)
    e = jnp.exp(logits - m)
    enc = e / jnp.sum(e, axis=1, keepdims=True)
    enc_ref[...] = enc

    q_ref[...] = jnp.dot(enc, cb, precision=jax.lax.Precision.HIGHEST,
                         preferred_element_type=jnp.float32)


def kernel(x, codebook, gumbel_noise):
    xf = x.reshape(B * N, D)
    gf = gumbel_noise.reshape(B * N, K)

    q, idx, enc = pl.pallas_call(
        _vq_tile,
        grid=(G,),
        in_specs=[
            pl.BlockSpec((T, D), lambda i: (i, 0)),
            pl.BlockSpec((K, D), lambda i: (0, 0)),
            pl.BlockSpec((T, K), lambda i: (i, 0)),
        ],
        out_specs=[
            pl.BlockSpec((T, D), lambda i: (i, 0)),
            pl.BlockSpec((1, 1, T), lambda i: (i, 0, 0)),
            pl.BlockSpec((T, K), lambda i: (i, 0)),
        ],
        out_shape=[
            jax.ShapeDtypeStruct((B * N, D), jnp.float32),
            jax.ShapeDtypeStruct((G, 1, T), jnp.int32),
            jax.ShapeDtypeStruct((B * N, K), jnp.float32),
        ],
        compiler_params=pltpu.CompilerParams(
            dimension_semantics=("arbitrary",)),
    )(xf, codebook, gf)

    return (q.reshape(B, N, D),
            idx.reshape(B, N),
            enc.reshape(B, N, K))


# speedup vs baseline: 8.3041x; 1.4365x over previous
import jax
import jax.numpy as jnp
from jax.experimental import pallas as pl
from jax.experimental.pallas import tpu as pltpu

B, N, D, K = 8, 576, 64, 512
T = 512
G = (B * N) // T


def _vq_tile(x_ref, cb_ref, g_ref, q_ref, idx_ref, enc_ref):
    x = x_ref[...]
    cb = cb_ref[...]
    g = g_ref[...]

    xn2 = jnp.sum(x * x, axis=1, keepdims=True)
    cn2 = jnp.sum(cb * cb, axis=1)[None, :]
    xc = jax.lax.dot_general(
        x, cb, (((1,), (1,)), ((), ())),
        precision=jax.lax.Precision.HIGHEST,
        preferred_element_type=jnp.float32)
    d2 = xn2 - 2.0 * xc + cn2
    d = jnp.sqrt(jnp.maximum(d2, 0.0))

    dmin = jnp.min(d, axis=1, keepdims=True)
    iota = jax.lax.broadcasted_iota(jnp.int32, (T, K), 1)
    idx = jnp.min(jnp.where(d == dmin, iota, K), axis=1)
    idx_ref[0, 0, :] = idx

    logits = g - d
    m = jnp.max(logits, axis=1, keepdims=True)
    e = jnp.exp(logits - m)
    enc = e / jnp.sum(e, axis=1, keepdims=True)
    enc_ref[...] = enc

    q_ref[...] = jnp.dot(enc, cb, preferred_element_type=jnp.float32)


def kernel(x, codebook, gumbel_noise):
    xf = x.reshape(B * N, D)
    gf = gumbel_noise.reshape(B * N, K)

    q, idx, enc = pl.pallas_call(
        _vq_tile,
        grid=(G,),
        in_specs=[
            pl.BlockSpec((T, D), lambda i: (i, 0)),
            pl.BlockSpec((K, D), lambda i: (0, 0)),
            pl.BlockSpec((T, K), lambda i: (i, 0)),
        ],
        out_specs=[
            pl.BlockSpec((T, D), lambda i: (i, 0)),
            pl.BlockSpec((1, 1, T), lambda i: (i, 0, 0)),
            pl.BlockSpec((T, K), lambda i: (i, 0)),
        ],
        out_shape=[
            jax.ShapeDtypeStruct((B * N, D), jnp.float32),
            jax.ShapeDtypeStruct((G, 1, T), jnp.int32),
            jax.ShapeDtypeStruct((B * N, K), jnp.float32),
        ],
        compiler_params=pltpu.CompilerParams(
            dimension_semantics=("parallel",)),
    )(xf, codebook, gf)

    return (q.reshape(B, N, D),
            idx.reshape(B, N),
            enc.reshape(B, N, K))


# speedup vs baseline: 8.5123x; 1.0251x over previous
import jax
import jax.numpy as jnp
from jax.experimental import pallas as pl
from jax.experimental.pallas import tpu as pltpu

B, N, D, K = 8, 576, 64, 512


def _vq_step(x_ref, cb_ref, g_ref, q_ref, idx_ref, enc_ref):
    b = pl.program_id(0)
    x = x_ref[0]
    cb = cb_ref[...]
    g = g_ref[0]

    xn2 = jnp.sum(x * x, axis=1, keepdims=True)
    cn2 = jnp.sum(cb * cb, axis=1)[None, :]
    xc = jax.lax.dot_general(
        x, cb, (((1,), (1,)), ((), ())),
        precision=jax.lax.Precision.HIGHEST,
        preferred_element_type=jnp.float32)
    d2 = xn2 - 2.0 * xc + cn2
    d = jnp.sqrt(jnp.maximum(d2, 0.0))

    dmin = jnp.min(d, axis=1, keepdims=True)
    iota = jax.lax.broadcasted_iota(jnp.int32, (N, K), 1)
    idx = jnp.min(jnp.where(d == dmin, iota, K), axis=1)
    idx_ref[b, :] = idx

    logits = g - d
    m = jnp.max(logits, axis=1, keepdims=True)
    e = jnp.exp(logits - m)
    enc = e / jnp.sum(e, axis=1, keepdims=True)
    enc_ref[0] = enc

    q_ref[0] = jnp.dot(enc, cb, preferred_element_type=jnp.float32)


def kernel(x, codebook, gumbel_noise):
    return pl.pallas_call(
        _vq_step,
        grid=(B,),
        in_specs=[
            pl.BlockSpec((1, N, D), lambda i: (i, 0, 0)),
            pl.BlockSpec((K, D), lambda i: (0, 0)),
            pl.BlockSpec((1, N, K), lambda i: (i, 0, 0)),
        ],
        out_specs=[
            pl.BlockSpec((1, N, D), lambda i: (i, 0, 0)),
            pl.BlockSpec((B, N), lambda i: (0, 0)),
            pl.BlockSpec((1, N, K), lambda i: (i, 0, 0)),
        ],
        out_shape=[
            jax.ShapeDtypeStruct((B, N, D), jnp.float32),
            jax.ShapeDtypeStruct((B, N), jnp.int32),
            jax.ShapeDtypeStruct((B, N, K), jnp.float32),
        ],
        compiler_params=pltpu.CompilerParams(
            dimension_semantics=("arbitrary",)),
    )(x, codebook, gumbel_noise)
